# baseline (speedup 1.0000x reference)
import functools

import jax
import jax.numpy as jnp
from jax import lax
from jax.experimental import pallas as pl
from jax.experimental.pallas import tpu as pltpu
from jax.experimental.pallas import tpu_sc as plsc

V, D, H, FF, L, E, TOPK = 32000, 768, 12, 2048, 2, 8, 2
B, S = 1, 2048
DH = D // H

_EPS = 1e-5

_NA = S * TOPK
_BMOE = 256
_NB = _NA // _BMOE + E
_NBPAD = 32
_NSLOT = _NB * _BMOE
_CAP = S


def _ln_rows(y, g, b):
    m = jnp.mean(y, axis=-1, keepdims=True)
    v = jnp.mean((y - m) ** 2, axis=-1, keepdims=True)
    return (y - m) * lax.rsqrt(v + _EPS) * g + b


_GROWS = 8


def _gather_kernel(*refs):
    out_ref = refs[-1]
    for j in range(_GROWS):
        out_ref[j, :] = refs[1 + j][0, 0, :]


def _gather_rows(table, idx, n_rows):
    t3 = table.reshape(table.shape[0], 1, D)
    grid_spec = pltpu.PrefetchScalarGridSpec(
        num_scalar_prefetch=1,
        grid=(n_rows // _GROWS,),
        in_specs=[
            pl.BlockSpec((1, 1, D), functools.partial(
                lambda j, i, ids: (ids[i * _GROWS + j], 0, 0), j))
            for j in range(_GROWS)
        ],
        out_specs=pl.BlockSpec((_GROWS, D), lambda i, ids: (i, 0)),
    )
    return pl.pallas_call(
        _gather_kernel,
        grid_spec=grid_spec,
        out_shape=jax.ShapeDtypeStruct((n_rows, D), jnp.float32),
    )(idx, *([t3] * _GROWS))


_NW = 32
_GCH = 64


def _sc_gather(table, idx, n_rows):
    per = n_rows // _NW

    def body(table_hbm, idx_hbm, out_hbm, idx_v, rows_v, sem):
        wid = lax.axis_index("s") * 2 + lax.axis_index("c")
        base = wid * per
        for t in range(per // _GCH):
            off = base + t * _GCH
            pltpu.sync_copy(idx_hbm.at[pl.ds(off, _GCH)], idx_v)
            pltpu.async_copy(table_hbm.at[idx_v], rows_v, sem).wait()
            pltpu.sync_copy(rows_v, out_hbm.at[pl.ds(off, _GCH)])

    k = functools.partial(
        pl.kernel,
        mesh=plsc.VectorSubcoreMesh(core_axis_name="c", subcore_axis_name="s"),
        out_type=jax.ShapeDtypeStruct((n_rows, D), jnp.float32),
        scratch_types=[
            pltpu.VMEM((_GCH,), jnp.int32),
            pltpu.VMEM((_GCH, D), jnp.float32),
            pltpu.SemaphoreType.DMA,
        ],
    )(body)
    return k(table, idx)


def _qkv_kernel(x_ref, w_ref, b_ref, o_ref):
    y = lax.dot_general(x_ref[...], w_ref[...], (((1,), (1,)), ((), ())),
                        preferred_element_type=jnp.float32) + b_ref[0]
    o_ref[...] = y[None]


def _qkv_proj(x, Wqkv, bqkv):
    BM = 512
    return pl.pallas_call(
        _qkv_kernel,
        grid=(S // BM, 3 * H),
        in_specs=[
            pl.BlockSpec((BM, D), lambda i, c: (i, 0)),
            pl.BlockSpec((DH, D), lambda i, c: (c, 0)),
            pl.BlockSpec((1, 1, DH), lambda i, c: (c, 0, 0)),
        ],
        out_specs=pl.BlockSpec((1, BM, DH), lambda i, c: (c, i, 0)),
        out_shape=jax.ShapeDtypeStruct((3 * H, S, DH), jnp.float32),
    )(x, Wqkv, bqkv.reshape(3 * H, 1, DH))


def _attn_kernel(q_ref, k_ref, v_ref, o_ref):
    q = q_ref[0]
    k = k_ref[0]
    s = lax.dot_general(q, k, (((1,), (1,)), ((), ())),
                        preferred_element_type=jnp.float32)
    s = s * (1.0 / (DH ** 0.5))
    m = jnp.max(s, axis=-1, keepdims=True)
    p = jnp.exp(s - m)
    denom = jnp.sum(p, axis=-1, keepdims=True)
    o = jnp.dot(p, v_ref[0], preferred_element_type=jnp.float32)
    o_ref[...] = (o / denom)[None]


def _attention(qkv):
    BM = 512
    return pl.pallas_call(
        _attn_kernel,
        grid=(H, S // BM),
        in_specs=[
            pl.BlockSpec((1, BM, DH), lambda h, i: (h, i, 0)),
            pl.BlockSpec((1, S, DH), lambda h, i: (H + h, 0, 0)),
            pl.BlockSpec((1, S, DH), lambda h, i: (2 * H + h, 0, 0)),
        ],
        out_specs=pl.BlockSpec((1, BM, DH), lambda h, i: (h, i, 0)),
        out_shape=jax.ShapeDtypeStruct((H, S, DH), jnp.float32),
    )(qkv, qkv, qkv)


def _oproj_ln_kernel(o_ref, w_ref, b_ref, r_ref, g_ref, bb_ref, out_ref):
    y = b_ref[...] + r_ref[...]
    for h in range(H):
        y = y + lax.dot_general(
            o_ref[h], w_ref[:, h, :], (((1,), (1,)), ((), ())),
            preferred_element_type=jnp.float32)
    out_ref[...] = _ln_rows(y, g_ref[...], bb_ref[...])


def _oproj_ln(o, Wo, bo, resid, g, b):
    BM = 512
    return pl.pallas_call(
        _oproj_ln_kernel,
        grid=(S // BM,),
        in_specs=[
            pl.BlockSpec((H, BM, DH), lambda i: (0, i, 0)),
            pl.BlockSpec((D, H, DH), lambda i: (0, 0, 0)),
            pl.BlockSpec((1, D), lambda i: (0, 0)),
            pl.BlockSpec((BM, D), lambda i: (i, 0)),
            pl.BlockSpec((1, D), lambda i: (0, 0)),
            pl.BlockSpec((1, D), lambda i: (0, 0)),
        ],
        out_specs=pl.BlockSpec((BM, D), lambda i: (i, 0)),
        out_shape=jax.ShapeDtypeStruct((S, D), jnp.float32),
    )(o, Wo.reshape(D, H, DH), bo.reshape(1, D), resid,
      g.reshape(1, D), b.reshape(1, D))


def _gate_kernel(x_ref, gw_ref, gb_ref, ti_ref, tp_ref):
    gs = lax.dot_general(x_ref[...], gw_ref[...], (((1,), (1,)), ((), ())),
                         preferred_element_type=jnp.float32) + gb_ref[...]
    n = gs.shape[0]
    ii = lax.broadcasted_iota(jnp.int32, (n, E), 1)
    a1 = jnp.argmax(gs, axis=-1).astype(jnp.int32)[:, None]
    m1 = jnp.max(gs, axis=-1, keepdims=True)
    gs2 = jnp.where(ii == a1, -jnp.inf, gs)
    a2 = jnp.argmax(gs2, axis=-1).astype(jnp.int32)[:, None]
    m2 = jnp.max(gs2, axis=-1, keepdims=True)
    p1 = 1.0 / (1.0 + jnp.exp(m2 - m1))
    p2 = 1.0 - p1
    ti_ref[...] = jnp.concatenate([a1, a2], axis=1)
    tp_ref[...] = jnp.concatenate([p1, p2], axis=1)


def _gate(x, gW, gb):
    BM = 1024
    return pl.pallas_call(
        _gate_kernel,
        grid=(S // BM,),
        in_specs=[
            pl.BlockSpec((BM, D), lambda i: (i, 0)),
            pl.BlockSpec((E, D), lambda i: (0, 0)),
            pl.BlockSpec((1, E), lambda i: (0, 0)),
        ],
        out_specs=[
            pl.BlockSpec((BM, TOPK), lambda i: (i, 0)),
            pl.BlockSpec((BM, TOPK), lambda i: (i, 0)),
        ],
        out_shape=[
            jax.ShapeDtypeStruct((S, TOPK), jnp.int32),
            jax.ShapeDtypeStruct((S, TOPK), jnp.float32),
        ],
    )(x, gW, gb.reshape(1, E))


def _route_body(ti_hbm, tp_hbm, disp_hbm, pslot_hbm, blk_hbm, pos0_hbm,
                pos1_hbm, ti_v, tp_v, dtmp, ptmp, dfin, pfin, pos0_v,
                pos1_v, blk_v, dlt_v):
    cid = lax.axis_index("c")
    sid = lax.axis_index("s")

    @pl.when((cid == 0) & (sid == 0))
    def _():
        pltpu.sync_copy(ti_hbm, ti_v)
        pltpu.sync_copy(tp_hbm, tp_v)
        lanes = lax.iota(jnp.int32, 16)

        def zf(c, carry):
            idx = c * 16 + lanes
            plsc.store_scatter(dfin, [idx], jnp.zeros(16, jnp.int32))
            plsc.store_scatter(pfin, [idx], jnp.zeros(16, jnp.float32))
            return carry
        lax.fori_loop(0, _NSLOT // 16, zf, jnp.int32(0))

        counts = []
        for e in range(E):
            def body(c, off, e=e):
                idx = c * 16 + lanes
                ids = plsc.load_gather(ti_v, [idx])
                pv = plsc.load_gather(tp_v, [idx])
                m = ids == e
                mi = m.astype(jnp.int32)
                ranks = plsc.cumsum(mi) - mi
                tot = jnp.sum(mi)
                slots = e * _CAP + off + ranks
                tok = lax.shift_right_logical(idx, 1)
                par = lax.bitwise_and(idx, 1)
                plsc.store_scatter(dtmp, [slots], tok, mask=m)
                plsc.store_scatter(ptmp, [slots], pv, mask=m)
                plsc.store_scatter(pos0_v, [tok], slots, mask=m & (par == 0))
                plsc.store_scatter(pos1_v, [tok], slots, mask=m & (par == 1))
                return off + tot
            counts.append(lax.fori_loop(0, _NA // 16, body, jnp.int32(0)))

        base = [jnp.int32(0)]
        for e in range(E):
            pad = ((counts[e] + (_BMOE - 1)) // _BMOE) * _BMOE
            base.append(base[e] + pad)
        for half in range(_NBPAD // 16):
            bidx = half * 16 + lanes
            row0 = bidx * _BMOE
            expv = jnp.zeros(16, jnp.int32)
            for e in range(E):
                inb = (row0 >= base[e]) & (row0 < base[e + 1])
                expv = jnp.where(inb, e, expv)
            expv = jnp.where(row0 < base[E], expv, E - 1)
            plsc.store_scatter(blk_v, [bidx], expv)

        for e in range(E):
            nch = (((counts[e] + (_BMOE - 1)) // _BMOE) * _BMOE) // 16

            def cp(c, carry, e=e):
                idx = c * 16 + lanes
                dv = plsc.load_gather(dtmp, [e * _CAP + idx])
                pv = plsc.load_gather(ptmp, [e * _CAP + idx])
                valid = idx < counts[e]
                dv = jnp.where(valid, dv, 0)
                pv = jnp.where(valid, pv, 0.0)
                plsc.store_scatter(dfin, [base[e] + idx], dv)
                plsc.store_scatter(pfin, [base[e] + idx], pv)
                return carry
            lax.fori_loop(0, nch, cp, jnp.int32(0))

        dl = jnp.zeros(16, jnp.int32)
        for e in range(E):
            dl = jnp.where(lanes == e, base[e] - e * _CAP, dl)
        plsc.store_scatter(dlt_v, [lanes], dl)

        def rm(c, carry):
            idx = c * 16 + lanes
            for arr in (pos0_v, pos1_v):
                v = plsc.load_gather(arr, [idx])
                ee = lax.shift_right_logical(v, 11)
                d = plsc.load_gather(dlt_v, [ee])
                plsc.store_scatter(arr, [idx], v + d)
            return carry
        lax.fori_loop(0, S // 16, rm, jnp.int32(0))

        pltpu.sync_copy(dfin, disp_hbm)
        pltpu.sync_copy(pfin, pslot_hbm)
        pltpu.sync_copy(blk_v, blk_hbm)
        pltpu.sync_copy(pos0_v, pos0_hbm)
        pltpu.sync_copy(pos1_v, pos1_hbm)


def _route_sc(ti_flat, tp_flat):
    k = functools.partial(
        pl.kernel,
        mesh=plsc.VectorSubcoreMesh(core_axis_name="c", subcore_axis_name="s"),
        compiler_params=pltpu.CompilerParams(needs_layout_passes=False),
        out_type=[
            jax.ShapeDtypeStruct((_NSLOT,), jnp.int32),
            jax.ShapeDtypeStruct((_NSLOT,), jnp.float32),
            jax.ShapeDtypeStruct((_NBPAD,), jnp.int32),
            jax.ShapeDtypeStruct((S,), jnp.int32),
            jax.ShapeDtypeStruct((S,), jnp.int32),
        ],
        scratch_types=[
            pltpu.VMEM((_NA,), jnp.int32),
            pltpu.VMEM((_NA,), jnp.float32),
            pltpu.VMEM((E * _CAP,), jnp.int32),
            pltpu.VMEM((E * _CAP,), jnp.float32),
            pltpu.VMEM((_NSLOT,), jnp.int32),
            pltpu.VMEM((_NSLOT,), jnp.float32),
            pltpu.VMEM((S,), jnp.int32),
            pltpu.VMEM((S,), jnp.int32),
            pltpu.VMEM((_NBPAD,), jnp.int32),
            pltpu.VMEM((16,), jnp.int32),
        ],
    )(_route_body)
    return k(ti_flat, tp_flat)


def _gmm_kernel(blk_ref, xg_ref, w1_ref, b1_ref, w2_ref, b2_ref, p_ref,
                yg_ref):
    del blk_ref
    h = lax.dot_general(xg_ref[...], w1_ref[0], (((1,), (1,)), ((), ())),
                        preferred_element_type=jnp.float32) + b1_ref[0]
    h = jnp.maximum(h, 0.0)
    y = lax.dot_general(h, w2_ref[0], (((1,), (1,)), ((), ())),
                        preferred_element_type=jnp.float32) + b2_ref[0]
    yg_ref[...] = y * p_ref[...]


def _grouped_moe(xg, W1, b1, W2, b2, pslot, blk):
    grid_spec = pltpu.PrefetchScalarGridSpec(
        num_scalar_prefetch=1,
        grid=(_NB,),
        in_specs=[
            pl.BlockSpec((_BMOE, D), lambda b, blk: (b, 0)),
            pl.BlockSpec((1, FF, D), lambda b, blk: (blk[b], 0, 0)),
            pl.BlockSpec((1, 1, FF), lambda b, blk: (blk[b], 0, 0)),
            pl.BlockSpec((1, D, FF), lambda b, blk: (blk[b], 0, 0)),
            pl.BlockSpec((1, 1, D), lambda b, blk: (blk[b], 0, 0)),
            pl.BlockSpec((_BMOE, 1), lambda b, blk: (b, 0)),
        ],
        out_specs=pl.BlockSpec((_BMOE, D), lambda b, blk: (b, 0)),
    )
    return pl.pallas_call(
        _gmm_kernel,
        grid_spec=grid_spec,
        out_shape=jax.ShapeDtypeStruct((_NSLOT, D), jnp.float32),
    )(blk, xg, W1, b1.reshape(E, 1, FF), W2, b2.reshape(E, 1, D),
      pslot.reshape(_NSLOT, 1))


_CROWS = 8


def _combine_kernel(pos0_ref, pos1_ref, *refs):
    del pos0_ref, pos1_ref
    a = refs[:_CROWS]
    bb = refs[_CROWS:2 * _CROWS]
    x_ref, g_ref, b_ref, out_ref = refs[2 * _CROWS:]
    rows = [a[j][0, 0, :] + bb[j][0, 0, :] for j in range(_CROWS)]
    y = x_ref[...] + jnp.concatenate([r[None] for r in rows], axis=0)
    out_ref[...] = _ln_rows(y, g_ref[...], b_ref[...])


def _combine_ln(yg, pos0, pos1, x, g, b):
    yg3 = yg.reshape(_NSLOT, 1, D)
    grid_spec = pltpu.PrefetchScalarGridSpec(
        num_scalar_prefetch=2,
        grid=(S // _CROWS,),
        in_specs=(
            [pl.BlockSpec((1, 1, D), functools.partial(
                lambda j, i, p0, p1: (p0[i * _CROWS + j], 0, 0), j))
             for j in range(_CROWS)]
            + [pl.BlockSpec((1, 1, D), functools.partial(
                lambda j, i, p0, p1: (p1[i * _CROWS + j], 0, 0), j))
               for j in range(_CROWS)]
            + [pl.BlockSpec((_CROWS, D), lambda i, p0, p1: (i, 0)),
               pl.BlockSpec((1, D), lambda i, p0, p1: (0, 0)),
               pl.BlockSpec((1, D), lambda i, p0, p1: (0, 0))]
        ),
        out_specs=pl.BlockSpec((_CROWS, D), lambda i, p0, p1: (i, 0)),
    )
    return pl.pallas_call(
        _combine_kernel,
        grid_spec=grid_spec,
        out_shape=jax.ShapeDtypeStruct((S, D), jnp.float32),
    )(pos0, pos1, *([yg3] * _CROWS), *([yg3] * _CROWS), x,
      g.reshape(1, D), b.reshape(1, D))


def _head_kernel(x_ref, g_ref, b_ref, w_ref, hb_ref, o_ref):
    xb = _ln_rows(x_ref[...], g_ref[...], b_ref[...])
    o_ref[...] = lax.dot_general(
        xb, w_ref[...], (((1,), (1,)), ((), ())),
        preferred_element_type=jnp.float32) + hb_ref[...]


def _head(x, lfg, lfb, hW, hb):
    BM, BN = 512, 1280
    return pl.pallas_call(
        _head_kernel,
        grid=(S // BM, V // BN),
        in_specs=[
            pl.BlockSpec((BM, D), lambda i, j: (i, 0)),
            pl.BlockSpec((1, D), lambda i, j: (0, 0)),
            pl.BlockSpec((1, D), lambda i, j: (0, 0)),
            pl.BlockSpec((BN, D), lambda i, j: (j, 0)),
            pl.BlockSpec((1, BN), lambda i, j: (0, j)),
        ],
        out_specs=pl.BlockSpec((BM, BN), lambda i, j: (i, j)),
        out_shape=jax.ShapeDtypeStruct((S, V), jnp.float32),
    )(x, lfg.reshape(1, D), lfb.reshape(1, D), hW, hb.reshape(1, V))


def kernel(input_ids, emb, Wqkv, bqkv, Wo, bo, gW, gb, W1, b1, W2, b2,
           n1g, n1b, n2g, n2b, lfg, lfb, hW, hb):
    ids = input_ids.reshape(S).astype(jnp.int32)
    x = _sc_gather(emb, ids, S)
    for l in range(L):
        qkv = _qkv_proj(x, Wqkv[l], bqkv[l])
        o = _attention(qkv)
        x = _oproj_ln(o, Wo[l], bo[l], x, n1g[l], n1b[l])
        ti, tp = _gate(x, gW[l], gb[l])
        disp, pslot, blk, pos0, pos1 = _route_sc(
            ti.reshape(_NA), tp.reshape(_NA))
        xg = _sc_gather(x, disp, _NSLOT)
        yg = _grouped_moe(xg, W1[l], b1[l], W2[l], b2[l], pslot, blk)
        x = _combine_ln(yg, pos0, pos1, x, n2g[l], n2b[l])
    out = _head(x, lfg, lfb, hW, hb)
    return out.reshape(B, S, V)
